# baseline (device time: 222587 ns/iter reference)
import jax
import jax.numpy as jnp
from jax import lax
from jax.experimental import pallas as pl
from jax.experimental.pallas import tpu as pltpu

N_DEV = 8
B_PER = 2
HQ_PER = 4
SQ = 256
SKV = 256
DH = 64
D_MODEL = 512


def kernel(x, Wq, K_ext, V_ext, Wo):
    wq_r = (Wq * 0.125).astype(jnp.bfloat16).reshape(
        D_MODEL, HQ_PER, DH).transpose(1, 0, 2)
    wo_r = Wo.astype(jnp.bfloat16).reshape(
        HQ_PER, DH, D_MODEL).transpose(0, 2, 1)
    w_pack = jnp.stack([wq_r, wo_r])

    def body(x_ref, w_ref, k_hbm, v_hbm, out_ref,
             comm, k_blk, v_blk, ksem, vsem, send_sem, recv_sem):
        my = lax.axis_index("i")
        right = lax.rem(my + 1, N_DEV)
        left = lax.rem(my + N_DEV - 1, N_DEV)
        b0 = my * B_PER

        comm[0] = w_ref[...]

        barrier = pltpu.get_barrier_semaphore()
        for nbr in (left, right):
            pl.semaphore_signal(barrier, inc=1, device_id=(nbr,),
                                device_id_type=pl.DeviceIdType.MESH)
        pl.semaphore_wait(barrier, 2)

        qi = lax.broadcasted_iota(jnp.int32, (SQ, SKV), 0)
        ki = lax.broadcasted_iota(jnp.int32, (SQ, SKV), 1)
        mask = (jnp.abs(qi - ki) <= 128) | (ki < 32) | (qi < 32)
        madd = jnp.where(mask, 0.0, -1e9).astype(jnp.float32)

        x2_bf = x_ref[...].reshape(B_PER * SQ, D_MODEL).astype(jnp.bfloat16)

        def start_kv(h):
            origin = lax.rem(my - h + 2 * N_DEV, N_DEV)
            copies = []
            for hh in range(HQ_PER):
                hidx = origin * HQ_PER + hh
                ck = pltpu.make_async_copy(
                    k_hbm.at[pl.ds(b0, B_PER), :, hidx, :],
                    k_blk.at[h % 2, :, hh], ksem.at[h % 2, hh])
                cv = pltpu.make_async_copy(
                    v_hbm.at[pl.ds(b0, B_PER), :, hidx, :],
                    v_blk.at[h % 2, :, hh], vsem.at[h % 2, hh])
                ck.start()
                cv.start()
                copies += [ck, cv]
            return copies

        kv_pending = start_kv(0)

        for h in range(N_DEV):
            if h < N_DEV - 1:
                rdma = pltpu.make_async_remote_copy(
                    src_ref=comm.at[h], dst_ref=comm.at[h + 1],
                    send_sem=send_sem.at[h], recv_sem=recv_sem.at[h],
                    device_id=(right,), device_id_type=pl.DeviceIdType.MESH)
                rdma.start()
                next_kv = start_kv(h + 1)

            for c in kv_pending:
                c.wait()

            q_bf = [
                lax.dot_general(
                    x2_bf, comm[h, 0, hh], (((1,), (0,)), ((), ())),
                    preferred_element_type=jnp.float32).astype(jnp.bfloat16)
                for hh in range(HQ_PER)
            ]

            for b in range(B_PER):
                acc = None
                for hh in range(HQ_PER):
                    kb = k_blk[h % 2, b, hh].astype(jnp.bfloat16)
                    vb = v_blk[h % 2, b, hh].astype(jnp.bfloat16)
                    s = lax.dot_general(
                        q_bf[hh][b * SQ:(b + 1) * SQ], kb,
                        (((1,), (1,)), ((), ())),
                        preferred_element_type=jnp.float32) + madd
                    e = jnp.exp(s)
                    w = (e * (1.0 / jnp.sum(e, axis=1, keepdims=True))
                         ).astype(jnp.bfloat16)
                    ctx = lax.dot_general(
                        w, vb, (((1,), (0,)), ((), ())),
                        preferred_element_type=jnp.float32)
                    contrib = lax.dot_general(
                        ctx.astype(jnp.bfloat16), comm[h, 1, hh],
                        (((1,), (1,)), ((), ())),
                        preferred_element_type=jnp.float32)
                    acc = contrib if acc is None else acc + contrib
                if h == 0:
                    out_ref[b] = acc
                else:
                    out_ref[b] = out_ref[b] + acc

            if h < N_DEV - 1:
                rdma.wait()
                kv_pending = next_kv

    return pl.pallas_call(
        body,
        out_shape=jax.ShapeDtypeStruct((B_PER, SQ, D_MODEL), jnp.float32),
        in_specs=[
            pl.BlockSpec(memory_space=pltpu.MemorySpace.VMEM),
            pl.BlockSpec(memory_space=pltpu.MemorySpace.VMEM),
            pl.BlockSpec(memory_space=pl.ANY),
            pl.BlockSpec(memory_space=pl.ANY),
        ],
        out_specs=pl.BlockSpec(memory_space=pltpu.MemorySpace.VMEM),
        scratch_shapes=[
            pltpu.MemorySpace.VMEM((N_DEV, 2, HQ_PER, D_MODEL, DH),
                                   jnp.bfloat16),
            pltpu.MemorySpace.VMEM((2, B_PER, HQ_PER, SKV, DH),
                                   jnp.float32),
            pltpu.MemorySpace.VMEM((2, B_PER, HQ_PER, SKV, DH),
                                   jnp.float32),
            pltpu.SemaphoreType.DMA((2, HQ_PER)),
            pltpu.SemaphoreType.DMA((2, HQ_PER)),
            pltpu.SemaphoreType.DMA((N_DEV - 1,)),
            pltpu.SemaphoreType.DMA((N_DEV - 1,)),
        ],
        compiler_params=pltpu.CompilerParams(collective_id=0),
    )(x, w_pack, K_ext, V_ext)


# device time: 126666 ns/iter; 1.7573x vs baseline; 1.7573x over previous
import os
import jax
import jax.numpy as jnp
from jax import lax
from jax.experimental import pallas as pl
from jax.experimental.pallas import tpu as pltpu

_PROBE = os.environ.get("PROBE", "")
N_DEV = 8
B_PER = 2
HQ_PER = 4
SQ = 256
SKV = 256
DH = 64
D_MODEL = 512
HK = HQ_PER * DH


def kernel(x, Wq, K_ext, V_ext, Wo):
    wq_bf = (Wq * 0.125).astype(jnp.bfloat16)
    wo_bf = Wo.T.astype(jnp.bfloat16)
    w_pack = jnp.stack([wq_bf, wo_bf])
    k2 = K_ext.reshape(2 * N_DEV, SKV, N_DEV * HK)
    v2 = V_ext.reshape(2 * N_DEV, SKV, N_DEV * HK)

    def body(x_ref, w_ref, k_hbm, v_hbm, out_ref,
             comm, k_blk, v_blk, ksem, vsem, send_sem, recv_sem):
        my = lax.axis_index("i")
        right = lax.rem(my + 1, N_DEV)
        left = lax.rem(my + N_DEV - 1, N_DEV)
        b0 = my * B_PER

        comm[0] = w_ref[...]

        barrier = pltpu.get_barrier_semaphore()
        for nbr in (left, right):
            pl.semaphore_signal(barrier, inc=1, device_id=(nbr,),
                                device_id_type=pl.DeviceIdType.MESH)
        pl.semaphore_wait(barrier, 2)

        qi = lax.broadcasted_iota(jnp.int32, (SQ, SKV), 0)
        ki = lax.broadcasted_iota(jnp.int32, (SQ, SKV), 1)
        mask = (jnp.abs(qi - ki) <= 128) | (ki < 32) | (qi < 32)
        madd = jnp.where(mask, 0.0, -1e9).astype(jnp.float32)

        x2_bf = x_ref[...].reshape(B_PER * SQ, D_MODEL).astype(jnp.bfloat16)
        if _PROBE == "comm":
            out_ref[...] = jnp.zeros_like(out_ref)

        def start_kv(h):
            origin = lax.rem(my - h + 2 * N_DEV, N_DEV)
            ck = pltpu.make_async_copy(
                k_hbm.at[pl.ds(b0, B_PER), :, pl.ds(origin * HK, HK)],
                k_blk.at[h % 2], ksem.at[h % 2])
            cv = pltpu.make_async_copy(
                v_hbm.at[pl.ds(b0, B_PER), :, pl.ds(origin * HK, HK)],
                v_blk.at[h % 2], vsem.at[h % 2])
            ck.start()
            cv.start()
            return ck, cv

        kv_pending = start_kv(0)

        for h in range(N_DEV):
            if h < N_DEV - 1 and _PROBE != "compute":
                rdma = pltpu.make_async_remote_copy(
                    src_ref=comm.at[h], dst_ref=comm.at[h + 1],
                    send_sem=send_sem.at[h], recv_sem=recv_sem.at[h],
                    device_id=(right,), device_id_type=pl.DeviceIdType.MESH)
                rdma.start()
            if h < N_DEV - 1:
                next_kv = start_kv(h + 1)

            for c in kv_pending:
                c.wait()

            hs = 0 if _PROBE == "compute" else h
            wq_o = comm[hs, 0]
            wo_o = comm[hs, 1]
            if _PROBE != "comm":
                q2 = lax.dot_general(
                    x2_bf, wq_o, (((1,), (0,)), ((), ())),
                    preferred_element_type=jnp.float32)
                q2_bf = q2.astype(jnp.bfloat16)

                ctx_rows = []
                for b in range(B_PER):
                    kb = k_blk[h % 2, b].astype(jnp.bfloat16)
                    vb = v_blk[h % 2, b].astype(jnp.bfloat16)
                    ctx_h = []
                    for hh in range(HQ_PER):
                        q_h = q2_bf[b * SQ:(b + 1) * SQ,
                                    hh * DH:(hh + 1) * DH]
                        s = lax.dot_general(
                            q_h, kb[:, hh * DH:(hh + 1) * DH],
                            (((1,), (1,)), ((), ())),
                            preferred_element_type=jnp.float32) + madd
                        e = jnp.exp(s)
                        w = (e * (1.0 / jnp.sum(e, axis=1, keepdims=True))
                             ).astype(jnp.bfloat16)
                        ctx_h.append(lax.dot_general(
                            w, vb[:, hh * DH:(hh + 1) * DH],
                            (((1,), (0,)), ((), ())),
                            preferred_element_type=jnp.float32))
                    ctx_rows.append(jnp.concatenate(ctx_h, axis=1))
                ctx2 = jnp.concatenate(ctx_rows, axis=0).astype(jnp.bfloat16)
                contrib = lax.dot_general(
                    ctx2, wo_o, (((1,), (1,)), ((), ())),
                    preferred_element_type=jnp.float32)
                contrib = contrib.reshape(B_PER, SQ, D_MODEL)
                if h == 0:
                    out_ref[...] = contrib
                else:
                    out_ref[...] = out_ref[...] + contrib

            if h < N_DEV - 1:
                if _PROBE != "compute":
                    rdma.wait()
                kv_pending = next_kv

    return pl.pallas_call(
        body,
        out_shape=jax.ShapeDtypeStruct((B_PER, SQ, D_MODEL), jnp.float32),
        in_specs=[
            pl.BlockSpec(memory_space=pltpu.MemorySpace.VMEM),
            pl.BlockSpec(memory_space=pltpu.MemorySpace.VMEM),
            pl.BlockSpec(memory_space=pl.ANY),
            pl.BlockSpec(memory_space=pl.ANY),
        ],
        out_specs=pl.BlockSpec(memory_space=pltpu.MemorySpace.VMEM),
        scratch_shapes=[
            pltpu.MemorySpace.VMEM((N_DEV, 2, D_MODEL, HK), jnp.bfloat16),
            pltpu.MemorySpace.VMEM((2, B_PER, SKV, HK), jnp.float32),
            pltpu.MemorySpace.VMEM((2, B_PER, SKV, HK), jnp.float32),
            pltpu.SemaphoreType.DMA((2,)),
            pltpu.SemaphoreType.DMA((2,)),
            pltpu.SemaphoreType.DMA((N_DEV - 1,)),
            pltpu.SemaphoreType.DMA((N_DEV - 1,)),
        ],
        compiler_params=pltpu.CompilerParams(collective_id=0),
    )(x, w_pack, k2, v2)


# device time: 94468 ns/iter; 2.3562x vs baseline; 1.3408x over previous
import os
import jax
import jax.numpy as jnp
from jax import lax
from jax.experimental import pallas as pl
from jax.experimental.pallas import tpu as pltpu

_PROBE = os.environ.get("PROBE", "")
N_DEV = 8
B_PER = 2
HQ_PER = 4
SQ = 256
SKV = 256
DH = 64
D_MODEL = 512
HK = HQ_PER * DH


def kernel(x, Wq, K_ext, V_ext, Wo):
    wq_bf = (Wq * 0.125).astype(jnp.bfloat16)
    wo_bf = Wo.T.astype(jnp.bfloat16)
    w_pack = jnp.stack([wq_bf, wo_bf])
    k2 = K_ext.reshape(2 * N_DEV, SKV, N_DEV * HK)
    v2 = V_ext.reshape(2 * N_DEV, SKV, N_DEV * HK)

    def body(x_ref, w_ref, k_hbm, v_hbm, out_ref,
             comm, k_blk, v_blk, ksem, vsem, send_sem, recv_sem):
        my = lax.axis_index("i")
        b0 = my * B_PER

        def ring_dev(p):
            p = lax.rem(p + 2 * N_DEV, N_DEV)
            return jnp.where(p < 4, p, 11 - p)

        my_pos = jnp.where(my < 4, my, 11 - my)
        right = ring_dev(my_pos + 1)
        left = ring_dev(my_pos - 1)

        comm[0] = w_ref[...]

        barrier = pltpu.get_barrier_semaphore()
        for nbr in (left, right):
            pl.semaphore_signal(barrier, inc=1, device_id=(nbr,),
                                device_id_type=pl.DeviceIdType.MESH)
        pl.semaphore_wait(barrier, 2)

        qi = lax.broadcasted_iota(jnp.int32, (SQ, SKV), 0)
        ki = lax.broadcasted_iota(jnp.int32, (SQ, SKV), 1)
        mask = (jnp.abs(qi - ki) <= 128) | (ki < 32) | (qi < 32)
        madd = jnp.where(mask, 0.0, -1e9).astype(jnp.float32)

        x2_bf = x_ref[...].reshape(B_PER * SQ, D_MODEL).astype(jnp.bfloat16)
        if _PROBE == "comm":
            out_ref[...] = jnp.zeros_like(out_ref)

        def start_kv(h):
            origin = ring_dev(my_pos - h)
            ck = pltpu.make_async_copy(
                k_hbm.at[pl.ds(b0, B_PER), :, pl.ds(origin * HK, HK)],
                k_blk.at[h % 2], ksem.at[h % 2])
            cv = pltpu.make_async_copy(
                v_hbm.at[pl.ds(b0, B_PER), :, pl.ds(origin * HK, HK)],
                v_blk.at[h % 2], vsem.at[h % 2])
            ck.start()
            cv.start()
            return ck, cv

        kv_pending = start_kv(0)

        for h in range(N_DEV):
            if h < N_DEV - 1 and _PROBE != "compute":
                rdma = pltpu.make_async_remote_copy(
                    src_ref=comm.at[h], dst_ref=comm.at[h + 1],
                    send_sem=send_sem.at[h], recv_sem=recv_sem.at[h],
                    device_id=(right,), device_id_type=pl.DeviceIdType.MESH)
                rdma.start()
            if h < N_DEV - 1:
                next_kv = start_kv(h + 1)

            for c in kv_pending:
                c.wait()

            hs = 0 if _PROBE == "compute" else h
            wq_o = comm[hs, 0]
            wo_o = comm[hs, 1]
            if _PROBE != "comm":
                q2 = lax.dot_general(
                    x2_bf, wq_o, (((1,), (0,)), ((), ())),
                    preferred_element_type=jnp.float32)
                q2_bf = q2.astype(jnp.bfloat16)

                ctx_rows = []
                for b in range(B_PER):
                    kb = k_blk[h % 2, b].astype(jnp.bfloat16)
                    vb = v_blk[h % 2, b].astype(jnp.bfloat16)
                    ctx_h = []
                    for hh in range(HQ_PER):
                        q_h = q2_bf[b * SQ:(b + 1) * SQ,
                                    hh * DH:(hh + 1) * DH]
                        s = lax.dot_general(
                            q_h, kb[:, hh * DH:(hh + 1) * DH],
                            (((1,), (1,)), ((), ())),
                            preferred_element_type=jnp.float32) + madd
                        e = jnp.exp(s)
                        w = (e * (1.0 / jnp.sum(e, axis=1, keepdims=True))
                             ).astype(jnp.bfloat16)
                        ctx_h.append(lax.dot_general(
                            w, vb[:, hh * DH:(hh + 1) * DH],
                            (((1,), (0,)), ((), ())),
                            preferred_element_type=jnp.float32))
                    ctx_rows.append(jnp.concatenate(ctx_h, axis=1))
                ctx2 = jnp.concatenate(ctx_rows, axis=0).astype(jnp.bfloat16)
                contrib = lax.dot_general(
                    ctx2, wo_o, (((1,), (1,)), ((), ())),
                    preferred_element_type=jnp.float32)
                contrib = contrib.reshape(B_PER, SQ, D_MODEL)
                if h == 0:
                    out_ref[...] = contrib
                else:
                    out_ref[...] = out_ref[...] + contrib

            if h < N_DEV - 1:
                if _PROBE != "compute":
                    rdma.wait()
                kv_pending = next_kv

    return pl.pallas_call(
        body,
        out_shape=jax.ShapeDtypeStruct((B_PER, SQ, D_MODEL), jnp.float32),
        in_specs=[
            pl.BlockSpec(memory_space=pltpu.MemorySpace.VMEM),
            pl.BlockSpec(memory_space=pltpu.MemorySpace.VMEM),
            pl.BlockSpec(memory_space=pl.ANY),
            pl.BlockSpec(memory_space=pl.ANY),
        ],
        out_specs=pl.BlockSpec(memory_space=pltpu.MemorySpace.VMEM),
        scratch_shapes=[
            pltpu.MemorySpace.VMEM((N_DEV, 2, D_MODEL, HK), jnp.bfloat16),
            pltpu.MemorySpace.VMEM((2, B_PER, SKV, HK), jnp.float32),
            pltpu.MemorySpace.VMEM((2, B_PER, SKV, HK), jnp.float32),
            pltpu.SemaphoreType.DMA((2,)),
            pltpu.SemaphoreType.DMA((2,)),
            pltpu.SemaphoreType.DMA((N_DEV - 1,)),
            pltpu.SemaphoreType.DMA((N_DEV - 1,)),
        ],
        compiler_params=pltpu.CompilerParams(collective_id=0),
    )(x, w_pack, k2, v2)
